# baseline (speedup 1.0000x reference)
import math
from functools import partial

import jax
import jax.numpy as jnp
from jax import lax
from jax.experimental import pallas as pl
from jax.experimental.pallas import tpu as pltpu

_EPS = 1e-5
_STATS_ROWS = 8


def _make_bands(w_hwio, width, dtype):
    KH, KW, Cin, Cout = w_hwio.shape
    pad = KH // 2
    idx = jnp.arange(width)
    rel = idx[:, None] - idx[None, :] + pad
    inband = (rel >= 0) & (rel < KW)
    taps = w_hwio[:, jnp.clip(rel, 0, KW - 1)]
    taps = jnp.where(inband[None, :, :, None, None], taps, 0.0)
    b = jnp.transpose(taps, (0, 1, 3, 2, 4))
    return b.reshape(KH, width * Cin, width * Cout).astype(dtype)


def _conv_taps(xb, band_ref, height):
    M = xb.shape[0]
    KH = band_ref.shape[0]
    pad = KH // 2
    hmod = lax.broadcasted_iota(jnp.int32, xb.shape, 0) % height
    acc = None
    for kh in range(KH):
        d = kh - pad
        if d == 0:
            lhs = xb
        else:
            lhs = pltpu.roll(xb, (-d) % M, 0)
            valid = hmod < (height - d) if d > 0 else hmod >= (-d)
            lhs = jnp.where(valid, lhs, jnp.zeros_like(lhs))
        part = jnp.dot(lhs, band_ref[kh], preferred_element_type=jnp.float32)
        acc = part if acc is None else acc + part
    return acc


def _write_stats(p_ref, acc):
    s = jnp.sum(acc, axis=0, keepdims=True)
    sq = jnp.sum(acc * acc, axis=0, keepdims=True)
    fill = jnp.zeros((p_ref.shape[1] - 2, acc.shape[1]), jnp.float32)
    p_ref[0, :, :] = jnp.concatenate([s, sq, fill], axis=0)


def _bn_coeffs(p_ref, g_ref, b_ref, cout, cnt):
    tot = jnp.sum(p_ref[:, 0, :], axis=0, keepdims=True)
    tsq = jnp.sum(p_ref[:, 1, :], axis=0, keepdims=True)
    wc = tot.shape[1]
    step = cout
    while step < wc:
        tot = tot + pltpu.roll(tot, step, 1)
        tsq = tsq + pltpu.roll(tsq, step, 1)
        step *= 2
    mean = tot / cnt
    var = jnp.maximum(tsq / cnt - mean * mean, 0.0)
    scale = g_ref[...] * lax.rsqrt(var + _EPS)
    shift = b_ref[...] - mean * scale
    return scale, shift


def _rows_bias(c_ref, height):
    nb, wc = c_ref.shape
    c = c_ref[...]
    return jnp.broadcast_to(c[:, None, :], (nb, height, wc)).reshape(nb * height, wc)


def _stage1_kernel(x_ref, band_ref, y_ref, p_ref, *, height):
    acc = _conv_taps(x_ref[...], band_ref, height)
    y_ref[...] = acc.astype(y_ref.dtype)
    _write_stats(p_ref, acc)


def _stage2_kernel(y1_ref, p1_ref, g_ref, b_ref, c_ref, band_ref, y2_ref,
                   p2_ref, *, height, cout, cnt):
    scale, shift = _bn_coeffs(p1_ref, g_ref, b_ref, cout, cnt)
    z = jnp.maximum(y1_ref[...].astype(jnp.float32) * scale + shift, 0.0)
    z = z + _rows_bias(c_ref, height)
    acc = _conv_taps(z.astype(jnp.bfloat16), band_ref, height)
    y2_ref[...] = acc.astype(y2_ref.dtype)
    _write_stats(p2_ref, acc)


def _stage3_kernel(y2_ref, p2_ref, g_ref, b_ref, c_ref, o_ref, *, height,
                   cout, cnt):
    scale, shift = _bn_coeffs(p2_ref, g_ref, b_ref, cout, cnt)
    z = jnp.maximum(y2_ref[...].astype(jnp.float32) * scale + shift, 0.0)
    o_ref[...] = z + _rows_bias(c_ref, height)


def kernel(x_nchw, condition, w1_hwio, w2_hwio, bn1_gamma, bn1_beta,
           bn2_gamma, bn2_beta, emb1_w, emb1_b, emb2_w, emb2_b):
    N, Cin, H, W = x_nchw.shape
    KH, KW, _, Cout = w1_hwio.shape
    assert KH == KW and KH % 2 == 1
    assert W & (W - 1) == 0, 'W must be a power of two for the lane all-reduce'
    WC_in, WC = W * Cin, W * Cout
    M = N * H
    cnt = float(M * W)
    inter = jnp.float32

    G = 16
    while N % G:
        G //= 2
    Nb = N // G
    Mb = Nb * H

    x2 = jnp.transpose(x_nchw, (0, 2, 3, 1)).reshape(M, WC_in).astype(jnp.bfloat16)

    band1 = _make_bands(w1_hwio, W, jnp.bfloat16)
    band2 = _make_bands(w2_hwio, W, jnp.bfloat16)

    g1t = jnp.tile(bn1_gamma, W).reshape(1, WC).astype(jnp.float32)
    b1t = jnp.tile(bn1_beta, W).reshape(1, WC).astype(jnp.float32)
    g2t = jnp.tile(bn2_gamma, W).reshape(1, WC).astype(jnp.float32)
    b2t = jnp.tile(bn2_beta, W).reshape(1, WC).astype(jnp.float32)

    c1 = jnp.tile(condition @ emb1_w.T + emb1_b, (1, W)).astype(jnp.float32)
    c2 = jnp.tile(condition @ emb2_w.T + emb2_b, (1, W)).astype(jnp.float32)

    row_spec_in = pl.BlockSpec((Mb, WC_in), lambda i: (i, 0))
    row_spec = pl.BlockSpec((Mb, WC), lambda i: (i, 0))
    stats_spec = pl.BlockSpec((1, _STATS_ROWS, WC), lambda i: (i, 0, 0))
    stats_full = pl.BlockSpec((G, _STATS_ROWS, WC), lambda i: (0, 0, 0))
    vec_spec = pl.BlockSpec((1, WC), lambda i: (0, 0))
    cond_spec = pl.BlockSpec((Nb, WC), lambda i: (i, 0))

    def band_spec(b):
        return pl.BlockSpec(b.shape, lambda i: (0, 0, 0))

    cparams = pltpu.CompilerParams(dimension_semantics=("parallel",))
    stats_shape = jax.ShapeDtypeStruct((G, _STATS_ROWS, WC), jnp.float32)

    y1, p1 = pl.pallas_call(
        partial(_stage1_kernel, height=H),
        grid=(G,),
        in_specs=[row_spec_in, band_spec(band1)],
        out_specs=[row_spec, stats_spec],
        out_shape=[jax.ShapeDtypeStruct((M, WC), inter), stats_shape],
        compiler_params=cparams,
    )(x2, band1)

    y2, p2 = pl.pallas_call(
        partial(_stage2_kernel, height=H, cout=Cout, cnt=cnt),
        grid=(G,),
        in_specs=[row_spec, stats_full, vec_spec, vec_spec, cond_spec,
                  band_spec(band2)],
        out_specs=[row_spec, stats_spec],
        out_shape=[jax.ShapeDtypeStruct((M, WC), inter), stats_shape],
        compiler_params=cparams,
    )(y1, p1, g1t, b1t, c1, band2)

    out = pl.pallas_call(
        partial(_stage3_kernel, height=H, cout=Cout, cnt=cnt),
        grid=(G,),
        in_specs=[row_spec, stats_full, vec_spec, vec_spec, cond_spec],
        out_specs=row_spec,
        out_shape=jax.ShapeDtypeStruct((M, WC), jnp.float32),
        compiler_params=cparams,
    )(y2, p2, g2t, b2t, c2)

    return jnp.transpose(out.reshape(N, H, W, Cout), (0, 3, 1, 2))
